# baseline (speedup 1.0000x reference)
import jax
import jax.numpy as jnp
from jax import lax
from jax.experimental import pallas as pl
from jax.experimental.pallas import tpu as pltpu

N = 4096
D = 256
BR = 128
NB = N // BR

_DNUMS = (((1,), (1,)), ((), ()))


def _dot_t(a, b):
    return lax.dot_general(a, b, _DNUMS, preferred_element_type=jnp.float32)


def _mega_kernel(x0_ref, t_ref, s_ref, o_ref, sb_ref, xb_ref, yb_ref, xf_ref, acc_ref):
    q = pl.program_id(0)
    i = pl.program_id(1)
    cols = pl.ds(i * BR, BR)

    @pl.when(q == 0)
    def _init():
        blk = x0_ref[...]
        xf_ref[:, cols] = blk
        acc_ref[:, cols] = blk
        xb_ref[:, cols] = blk.astype(jnp.bfloat16)

    @pl.when(q % 2 == 1)
    def _t_phase():
        yb_ref[:, cols] = _dot_t(
            xb_ref[...], t_ref[...].astype(jnp.bfloat16)
        ).astype(jnp.bfloat16)

    @pl.when(q == 2)
    def _s_load():
        sb_ref[cols, :] = s_ref[...].astype(jnp.bfloat16)

    @pl.when((q == 2) | (q == 4) | (q == 6))
    def _s_phase():
        st = sb_ref[cols, :]
        xn = _dot_t(yb_ref[...], st) + xf_ref[:, cols]
        an = acc_ref[:, cols] + xn

        @pl.when(q != 6)
        def _():
            xf_ref[:, cols] = xn
            xb_ref[:, cols] = xn.astype(jnp.bfloat16)
            acc_ref[:, cols] = an

        @pl.when(q == 6)
        def _():
            o_ref[...] = 0.25 * an


def _x0_idx(q, i):
    return (0, jnp.where(q == 0, i, NB - 1))


def _t_idx(q, i):
    return (jnp.where(q % 2 == 1, i, jnp.where(q == 0, 0, NB - 1)), 0)


def _s_idx(q, i):
    return (jnp.where(q == 2, i, jnp.where(q < 2, 0, NB - 1)), 0)


def _o_idx(q, i):
    return (0, jnp.where(q == 6, i, 0))


def kernel(pois_embs, HG_poi_src, HG_poi_tar):
    x0t = pois_embs.T
    outt = pl.pallas_call(
        _mega_kernel,
        grid=(7, NB),
        in_specs=[
            pl.BlockSpec((D, BR), _x0_idx),
            pl.BlockSpec((BR, N), _t_idx),
            pl.BlockSpec((BR, N), _s_idx),
        ],
        out_specs=pl.BlockSpec((D, BR), _o_idx),
        out_shape=jax.ShapeDtypeStruct((D, N), jnp.float32),
        scratch_shapes=[
            pltpu.VMEM((N, N), jnp.bfloat16),
            pltpu.VMEM((D, N), jnp.bfloat16),
            pltpu.VMEM((D, N), jnp.bfloat16),
            pltpu.VMEM((D, N), jnp.float32),
            pltpu.VMEM((D, N), jnp.float32),
        ],
        compiler_params=pltpu.CompilerParams(
            dimension_semantics=("arbitrary", "arbitrary"),
        ),
    )(x0t, HG_poi_tar, HG_poi_src)
    return outt.T


# speedup vs baseline: 1.3874x; 1.3874x over previous
import jax
import jax.numpy as jnp
from jax.experimental import pallas as pl
from jax.experimental.pallas import tpu as pltpu

N = 4096
D = 256
BR = 256


def _mm2_kernel(a1_ref, a2_ref, x_ref, o_ref):
    o_ref[:BR, :] = jnp.dot(
        a1_ref[...].astype(jnp.bfloat16), x_ref[...],
        preferred_element_type=jnp.float32)
    o_ref[BR:, :] = jnp.dot(
        a2_ref[...].astype(jnp.bfloat16), x_ref[...],
        preferred_element_type=jnp.float32)


def _mm2_add_kernel(a1_ref, a2_ref, x_ref, r_ref, o_ref):
    o_ref[:BR, :] = jnp.dot(
        a1_ref[...].astype(jnp.bfloat16), x_ref[...],
        preferred_element_type=jnp.float32) + r_ref[:BR, :]
    o_ref[BR:, :] = jnp.dot(
        a2_ref[...].astype(jnp.bfloat16), x_ref[...],
        preferred_element_type=jnp.float32) + r_ref[BR:, :]


def _mm2_mean_kernel(a1_ref, a2_ref, x_ref, x0_ref, x1_ref, x2_ref, o_ref):
    d1 = jnp.dot(a1_ref[...].astype(jnp.bfloat16), x_ref[...],
                 preferred_element_type=jnp.float32)
    d2 = jnp.dot(a2_ref[...].astype(jnp.bfloat16), x_ref[...],
                 preferred_element_type=jnp.float32)
    d = jnp.concatenate([d1, d2], axis=0)
    o_ref[...] = 0.25 * (x0_ref[...] + x1_ref[...] + d) + 0.5 * x2_ref[...]


_a1_spec = pl.BlockSpec((BR, N), lambda i: (2 * i, 0))
_a2_spec = pl.BlockSpec((BR, N), lambda i: (2 * i + 1, 0))
_full_spec = pl.BlockSpec((N, D), lambda i: (0, 0))
_out_spec = pl.BlockSpec((2 * BR, D), lambda i: (i, 0))
_params = pltpu.CompilerParams(dimension_semantics=("arbitrary",))
_GRID = (N // (2 * BR),)


def _mm(a, x):
    return pl.pallas_call(
        _mm2_kernel,
        grid=_GRID,
        in_specs=[_a1_spec, _a2_spec, _full_spec],
        out_specs=_out_spec,
        out_shape=jax.ShapeDtypeStruct((N, D), jnp.float32),
        compiler_params=_params,
    )(a, a, x)


def _mm_add(a, x, r):
    return pl.pallas_call(
        _mm2_add_kernel,
        grid=_GRID,
        in_specs=[_a1_spec, _a2_spec, _full_spec, _out_spec],
        out_specs=_out_spec,
        out_shape=jax.ShapeDtypeStruct((N, D), jnp.float32),
        compiler_params=_params,
    )(a, a, x, r)


def _mm_mean(a, x, x0, x1, x2):
    return pl.pallas_call(
        _mm2_mean_kernel,
        grid=_GRID,
        in_specs=[_a1_spec, _a2_spec, _full_spec, _out_spec, _out_spec, _out_spec],
        out_specs=_out_spec,
        out_shape=jax.ShapeDtypeStruct((N, D), jnp.float32),
        compiler_params=_params,
    )(a, a, x, x0, x1, x2)


def kernel(pois_embs, HG_poi_src, HG_poi_tar):
    x0 = pois_embs
    x0b = x0.astype(jnp.bfloat16)

    y1 = _mm(HG_poi_tar, x0b)
    x1 = _mm_add(HG_poi_src, y1.astype(jnp.bfloat16), x0)

    y2 = _mm(HG_poi_tar, x1.astype(jnp.bfloat16))
    x2 = _mm_add(HG_poi_src, y2.astype(jnp.bfloat16), x1)

    y3 = _mm(HG_poi_tar, x2.astype(jnp.bfloat16))
    return _mm_mean(HG_poi_src, y3.astype(jnp.bfloat16), x0, x1, x2)
